# Initial kernel scaffold; baseline (speedup 1.0000x reference)
import jax
import jax.numpy as jnp
from jax.experimental import pallas as pl


def kernel(flat, cu_seqlens, W1, b1, W2, b2):
    raise NotImplementedError("write your pallas kernel here")


# speedup vs baseline: 2.2865x; 2.2865x over previous
import functools

import jax
import jax.numpy as jnp
from jax import lax
from jax.experimental import pallas as pl
from jax.experimental.pallas import tpu as pltpu
from jax.experimental.pallas import tpu_sc as plsc

B = 16
MAX_LEN = 2048
TOTAL = 16384
D = 512

TILE = 512
N_TILES = TOTAL // TILE
PAD_ROWS = 40
P_PAD = PAD_ROWS * D
ZERO_SLOT = TOTAL + 512

N_WORKERS = 32
OUT_ELEMS = B * MAX_LEN
CHUNK = OUT_ELEMS // N_WORKERS


def _matmul_body(x_ref, w1_ref, b1_ref, w2t_ref, b2_ref, s_ref):
    h = jnp.dot(x_ref[...], w1_ref[...], preferred_element_type=jnp.float32)
    h = jnp.maximum(h + b1_ref[...], 0.0)
    s = jnp.sum(h * w2t_ref[...], axis=1, keepdims=True) + b2_ref[0, 0]
    s_ref[...] = s


def _softmax_body(cu_ref, s_ref, out_ref):
    s = s_ref[...]
    row = lax.broadcasted_iota(jnp.int32, (N_TILES, TILE), 0)
    col = lax.broadcasted_iota(jnp.int32, (N_TILES, TILE), 1)
    t = row * TILE + col
    neg_inf = jnp.float32(-jnp.inf)

    masks = []
    m_tok = jnp.zeros((N_TILES, TILE), jnp.float32)
    for b in range(B):
        mask = (t >= cu_ref[b]) & (t < cu_ref[b + 1])
        masks.append(mask)
        m_b = jnp.max(jnp.where(mask, s, neg_inf))
        m_tok = m_tok + jnp.where(mask, m_b, 0.0)

    e = jnp.exp(s - m_tok)
    d_tok = jnp.ones((N_TILES, TILE), jnp.float32)
    for b in range(B):
        sum_b = jnp.sum(jnp.where(masks[b], e, 0.0))
        d_tok = d_tok + jnp.where(masks[b], sum_b - 1.0, 0.0)

    out_ref[0:N_TILES, :] = e / d_tok
    out_ref[N_TILES:PAD_ROWS, :] = jnp.zeros((PAD_ROWS - N_TILES, TILE),
                                             jnp.float32)


def _sc_gather_body(p_hbm, idx_hbm, out_hbm, p_v, idx_v, out_v):
    wid = lax.axis_index("s") * 2 + lax.axis_index("c")
    base = wid * CHUNK
    pltpu.sync_copy(p_hbm, p_v)
    pltpu.sync_copy(idx_hbm.at[pl.ds(base, CHUNK)], idx_v)

    def body(j, carry):
        iv = idx_v[pl.ds(j * 16, 16)]
        out_v[pl.ds(j * 16, 16)] = plsc.load_gather(p_v, [iv])
        return carry

    lax.fori_loop(0, CHUNK // 16, body, 0)
    pltpu.sync_copy(out_v, out_hbm.at[pl.ds(base, CHUNK)])


def kernel(flat, cu_seqlens, W1, b1, W2, b2):
    cu = cu_seqlens.astype(jnp.int32)
    b1r = b1.reshape(1, D)
    w2t = W2.reshape(1, D)
    b2r = b2.reshape(1, 1)

    scores = pl.pallas_call(
        _matmul_body,
        grid=(N_TILES,),
        in_specs=[
            pl.BlockSpec((TILE, D), lambda i: (i, 0)),
            pl.BlockSpec((D, D), lambda i: (0, 0)),
            pl.BlockSpec((1, D), lambda i: (0, 0)),
            pl.BlockSpec((1, D), lambda i: (0, 0)),
            pl.BlockSpec((1, 1), lambda i: (0, 0)),
        ],
        out_specs=pl.BlockSpec((TILE, 1), lambda i: (i, 0)),
        out_shape=jax.ShapeDtypeStruct((TOTAL, 1), jnp.float32),
    )(flat, W1, b1r, w2t, b2r)

    probs = pl.pallas_call(
        _softmax_body,
        grid_spec=pltpu.PrefetchScalarGridSpec(
            num_scalar_prefetch=1,
            grid=(1,),
            in_specs=[pl.BlockSpec((N_TILES, TILE), lambda i, cu_ref: (0, 0))],
            out_specs=pl.BlockSpec((PAD_ROWS, TILE),
                                   lambda i, cu_ref: (0, 0)),
        ),
        out_shape=jax.ShapeDtypeStruct((PAD_ROWS, TILE), jnp.float32),
    )(cu, scores.reshape(N_TILES, TILE))

    k = jnp.arange(OUT_ELEMS, dtype=jnp.int32)
    seg = k >> 11
    pos = k & (MAX_LEN - 1)
    starts = cu[seg]
    lens = cu[seg + 1] - starts
    idx = jnp.where(pos < lens, starts + pos, ZERO_SLOT)

    sc_gather = functools.partial(
        pl.kernel,
        mesh=plsc.VectorSubcoreMesh(core_axis_name="c", subcore_axis_name="s"),
        out_type=jax.ShapeDtypeStruct((OUT_ELEMS,), jnp.float32),
        scratch_types=[
            pltpu.VMEM((P_PAD,), jnp.float32),
            pltpu.VMEM((CHUNK,), jnp.int32),
            pltpu.VMEM((CHUNK,), jnp.float32),
        ],
        compiler_params=pltpu.CompilerParams(needs_layout_passes=False),
    )(_sc_gather_body)

    dense = sc_gather(probs.reshape(P_PAD), idx)
    return dense.reshape(B, MAX_LEN)
